# baseline (speedup 1.0000x reference)
import functools

import jax
import jax.numpy as jnp
from jax import lax
from jax.experimental import pallas as pl
from jax.experimental.pallas import tpu as pltpu
from jax.experimental.pallas import tpu_sc as plsc

_NC = 2
_NS = 16
_NW = _NC * _NS


@functools.partial(jax.jit, static_argnums=(2, 3, 4, 5, 6))
def _emb_gather(x_flat, table, B, D, C, NBUF, K):
    b_per_w = B // _NW
    n_chunks = b_per_w // C
    assert n_chunks * C == b_per_w
    assert n_chunks % NBUF == 0 and NBUF > K >= 1
    mesh = plsc.VectorSubcoreMesh(core_axis_name="c", subcore_axis_name="s")

    @functools.partial(
        pl.kernel,
        out_type=jax.ShapeDtypeStruct((B, D), jnp.float32),
        mesh=mesh,
        scratch_types=[
            pltpu.VMEM((NBUF, C), jnp.int32),
            pltpu.VMEM((NBUF, C, D), jnp.float32),
            pltpu.SemaphoreType.DMA((NBUF,)),
            pltpu.SemaphoreType.DMA((NBUF,)),
        ],
        compiler_params=pltpu.CompilerParams(use_tc_tiling_on_sc=False),
    )
    def k(x_hbm, table_hbm, out_hbm, idx_v, rows_v, s_g, s_st):
        wid = lax.axis_index("s") * _NC + lax.axis_index("c")
        base = wid * b_per_w

        def gather_copy(b):
            return pltpu.make_async_copy(
                table_hbm.at[idx_v.at[b]], rows_v.at[b], s_g.at[b])

        def store_copy(g, b):
            return pltpu.make_async_copy(
                rows_v.at[b], out_hbm.at[pl.ds(base + g * C, C)], s_st.at[b])

        def outer(g2, carry):
            for j in range(NBUF):
                g = g2 * NBUF + j

                @pl.when(g >= NBUF)
                def _():
                    store_copy(g - NBUF, j).wait()

                pltpu.sync_copy(x_hbm.at[pl.ds(base + g * C, C)],
                                idx_v.at[j])
                gather_copy(j).start()

                jd = (j - K) % NBUF

                @pl.when(g >= K)
                def _():
                    gather_copy(jd).wait()
                    store_copy(g - K, jd).start()

            return carry

        lax.fori_loop(0, n_chunks // NBUF, outer, 0)

        for c in range(n_chunks - K, n_chunks):
            b = c % NBUF
            gather_copy(b).wait()
            store_copy(c, b).start()
        for c in range(n_chunks - NBUF, n_chunks):
            store_copy(c, c % NBUF).wait()

    return k(x_flat, table)


def kernel(x, table):
    B = x.shape[0] * x.shape[1]
    D = table.shape[1]
    out = _emb_gather(x.reshape(B).astype(jnp.int32), table, B, D,
                      512, 4, 2)
    return out.reshape(x.shape[0], x.shape[1], D)
